# baseline (speedup 1.0000x reference)
import jax
import jax.numpy as jnp
from jax.experimental import pallas as pl
from jax.experimental.pallas import tpu as pltpu

_NB = 16
_CHUNK = 2048


def _body(x_hbm, w_ref, out_ref, xbuf, sems):
    n_chunks = x_hbm.shape[0] // _CHUNK
    w = w_ref[...]
    c_total = w.shape[1]
    v_total = c_total // _NB
    c = jax.lax.broadcasted_iota(jnp.int32, (v_total, c_total), 1)
    v = jax.lax.broadcasted_iota(jnp.int32, (v_total, c_total), 0)
    pow2 = jnp.left_shift(jnp.int32(1), c % _NB).astype(jnp.float32)
    packT = jnp.where(c // _NB == v, pow2, 0.0).astype(jnp.bfloat16)

    def copy(i):
        return pltpu.make_async_copy(
            x_hbm.at[pl.ds(i * _CHUNK, _CHUNK), :],
            xbuf.at[i % 2],
            sems.at[i % 2],
        )

    copy(0).start()
    for i in range(n_chunks):
        if i + 1 < n_chunks:
            copy(i + 1).start()
        copy(i).wait()
        x = xbuf[i % 2]
        proj = jnp.dot(x, w, preferred_element_type=jnp.float32)
        bits = (proj > 0).astype(jnp.bfloat16)
        votes_t = jax.lax.dot_general(
            packT, bits, (((1,), (1,)), ((), ())),
            preferred_element_type=jnp.float32)
        out_ref[:, i * _CHUNK:(i + 1) * _CHUNK] = votes_t.astype(jnp.int32)


def kernel(x, W):
    Q, D = x.shape
    V, _, B = W.shape
    W2 = jnp.transpose(W, (1, 0, 2)).reshape(D, V * B)
    return pl.pallas_call(
        _body,
        in_specs=[
            pl.BlockSpec(memory_space=pltpu.MemorySpace.HBM),
            pl.BlockSpec(memory_space=pltpu.MemorySpace.VMEM),
        ],
        out_specs=pl.BlockSpec(memory_space=pltpu.MemorySpace.VMEM),
        out_shape=jax.ShapeDtypeStruct((V, Q), jnp.int32),
        scratch_shapes=[
            pltpu.VMEM((2, _CHUNK, D), jnp.float32),
            pltpu.SemaphoreType.DMA((2,)),
        ],
    )(x, W2)


# speedup vs baseline: 1.0400x; 1.0400x over previous
import jax
import jax.numpy as jnp
from jax.experimental import pallas as pl
from jax.experimental.pallas import tpu as pltpu

_NB = 16
_CHUNK = 2048


def _body(x_hbm, w_ref, out_ref, xbuf, sems):
    n_chunks = x_hbm.shape[0] // _CHUNK
    w = w_ref[...]
    c_total = w.shape[1]
    v_total = c_total // _NB
    c = jax.lax.broadcasted_iota(jnp.int32, (v_total, c_total), 1)
    v = jax.lax.broadcasted_iota(jnp.int32, (v_total, c_total), 0)
    pow2 = jnp.left_shift(jnp.int32(1), c % _NB).astype(jnp.float32)
    packT = jnp.where(c // _NB == v, pow2, 0.0).astype(jnp.bfloat16)

    def copy(i):
        return pltpu.make_async_copy(
            x_hbm.at[pl.ds(i * _CHUNK, _CHUNK), :],
            xbuf.at[i % 4],
            sems.at[i % 4],
        )

    depth = 3
    for j in range(depth):
        copy(j).start()
    for i in range(n_chunks):
        if i + depth < n_chunks:
            copy(i + depth).start()
        copy(i).wait()
        x = xbuf[i % 4]
        proj = jnp.dot(x, w, preferred_element_type=jnp.float32)
        bits = (proj > 0).astype(jnp.bfloat16)
        votes_t = jax.lax.dot_general(
            packT, bits, (((1,), (1,)), ((), ())),
            preferred_element_type=jnp.float32)
        out_ref[:, i * _CHUNK:(i + 1) * _CHUNK] = votes_t.astype(jnp.int32)


def kernel(x, W):
    Q, D = x.shape
    V, _, B = W.shape
    W2 = jnp.transpose(W, (1, 0, 2)).reshape(D, V * B)
    return pl.pallas_call(
        _body,
        in_specs=[
            pl.BlockSpec(memory_space=pltpu.MemorySpace.HBM),
            pl.BlockSpec(memory_space=pltpu.MemorySpace.VMEM),
        ],
        out_specs=pl.BlockSpec(memory_space=pltpu.MemorySpace.VMEM),
        out_shape=jax.ShapeDtypeStruct((V, Q), jnp.int32),
        scratch_shapes=[
            pltpu.VMEM((4, _CHUNK, D), jnp.float32),
            pltpu.SemaphoreType.DMA((4,)),
        ],
    )(x, W2)
